# Initial kernel scaffold; baseline (speedup 1.0000x reference)
import jax
import jax.numpy as jnp
from jax.experimental import pallas as pl


def kernel(x_pfc, x_vtx, batch_pfc, batch_vtx, P1, pb1, P2, pb2, V1, vb1, V2, vb2, C1, cb1, O1, ob1, O2, ob2, O3, ob3, O4, ob4):
    raise NotImplementedError("write your pallas kernel here")


# speedup vs baseline: 9.0310x; 9.0310x over previous
import functools
import jax
import jax.numpy as jnp
from jax import lax
from jax.experimental import pallas as pl
from jax.experimental.pallas import tpu as pltpu

F32 = jnp.float32
BIG1 = 1e10
BIG2 = 4e10
NEG = -1e30
HI = jax.lax.Precision.HIGHEST
K = 16


def _lr(x):
    return jnp.where(x > 0, x, 0.01 * x)


def _prep_body(xp_ref, xv_ref, P1_ref, pb1_ref, P2_ref, pb2_ref,
               V1_ref, vb1_ref, V2_ref, vb2_ref, Ep_ref, Ev_ref):
    xp = xp_ref[...]
    xv = xv_ref[...]
    Ep_ref[...] = _lr(jnp.dot(_lr(jnp.dot(xp, P1_ref[...]) + pb1_ref[...]),
                              P2_ref[...]) + pb2_ref[...])
    Ev_ref[...] = _lr(jnp.dot(_lr(jnp.dot(xv, V1_ref[...]) + vb1_ref[...]),
                              V2_ref[...]) + vb2_ref[...])


def _conv_core(A, bt, wlo, whi, b_ref, bs_ref, c1a, c1b, cb, strip_ref, BT):
    n_t = jnp.sum(A * A, axis=1, keepdims=True)
    ones_row = jnp.ones((1, 32), F32)
    ui = jnp.dot(A, c1a, preferred_element_type=F32)

    def p1(j, _):
        idx = pl.multiple_of(j * 128, 128)
        S_tile = b_ref[pl.ds(idx, 128), :]
        D0 = lax.dot_general(A, S_tile, (((1,), (1,)), ((), ())),
                             preferred_element_type=F32)
        ns = lax.dot_general(ones_row, S_tile * S_tile,
                             (((1,), (1,)), ((), ())),
                             precision=HI, preferred_element_type=F32)
        D = (n_t - 2.0 * D0) + ns
        bs = bs_ref[:, pl.ds(idx, 128)]
        D = jnp.where(bt != bs, BIG1, D)
        strip_ref[:, pl.ds(idx, 128)] = D
        return 0

    lax.fori_loop(wlo, whi, p1, 0)

    def sweep(s, carry):
        m_prev, acc = carry

        def inner(j, c):
            cm, xj = c
            idx = pl.multiple_of(j * 128, 128)
            t = strip_ref[:, pl.ds(idx, 128)]
            oh = t == m_prev
            t = jnp.where(oh, BIG2, t)
            strip_ref[:, pl.ds(idx, 128)] = t
            S_tile = b_ref[pl.ds(idx, 128), :]
            xj = xj + lax.dot_general(oh.astype(F32), S_tile,
                                      (((1,), (0,)), ((), ())),
                                      precision=HI, preferred_element_type=F32)
            return (jnp.minimum(cm, jnp.min(t, axis=1, keepdims=True)), xj)

        cm, XJ = lax.fori_loop(wlo, whi, inner,
                               (jnp.full((BT, 1), 1e30, F32),
                                jnp.zeros((BT, 32), F32)))
        msg = _lr((ui + jnp.dot(XJ - A, c1b, preferred_element_type=F32)) + cb)
        acc = jnp.where(s > 0, jnp.maximum(acc, msg), acc)
        return (cm, acc)

    _, acc = lax.fori_loop(0, K + 1, sweep,
                           (jnp.full((BT, 1), 3e30, F32),
                            jnp.full((BT, 32), NEG, F32)))
    return acc


def _conv1_body(wlo_ref, whi_ref, a_ref, bt_ref, b_ref, bs_ref,
                C1_ref, cb_ref, F_ref, strip_ref, *, BT):
    pid = pl.program_id(0)
    F_ref[...] = _conv_core(a_ref[...], bt_ref[...],
                            wlo_ref[pid], whi_ref[pid], b_ref, bs_ref,
                            C1_ref[0:32, :], C1_ref[32:64, :], cb_ref[...],
                            strip_ref, BT)


def _conv2_body(wlo_ref, whi_ref, a_ref, bt_ref, b_ref, bs_ref, C1_ref, cb_ref,
                O1_ref, ob1_ref, O2_ref, ob2_ref, O3_ref, ob3_ref,
                O4_ref, ob4_ref, out_ref, strip_ref, *, BT):
    pid = pl.program_id(0)
    feats = _conv_core(a_ref[...], bt_ref[...],
                       wlo_ref[pid], whi_ref[pid], b_ref, bs_ref,
                       C1_ref[0:32, :], C1_ref[32:64, :], cb_ref[...],
                       strip_ref, BT)
    h = _lr(jnp.dot(feats, O1_ref[...]) + ob1_ref[...])
    h = _lr(jnp.dot(h, O2_ref[...]) + ob2_ref[...])
    h = _lr(jnp.dot(h, O3_ref[...]) + ob3_ref[...])
    out_ref[...] = _lr(jnp.dot(h, O4_ref[...]) + ob4_ref[...])


def _windows(batch_t, batch_s, BT, n_blk_s):
    i0 = jnp.arange(batch_t.shape[0] // BT) * BT
    blo = batch_t[i0]
    bhi = batch_t[i0 + BT - 1]
    lo = jnp.searchsorted(batch_s, blo, side='left')
    hic = jnp.searchsorted(batch_s, bhi, side='right')
    wlo = (lo // 128).astype(jnp.int32)
    whi = jnp.minimum((hic + 127) // 128, n_blk_s).astype(jnp.int32)
    whi = jnp.maximum(whi, wlo)
    return wlo, whi


def kernel(x_pfc, x_vtx, batch_pfc, batch_vtx,
           P1, pb1, P2, pb2, V1, vb1, V2, vb2, C1, cb1,
           O1, ob1, O2, ob2, O3, ob3, O4, ob4):
    N = x_pfc.shape[0]
    M = x_vtx.shape[0]
    BT = 512 if N % 512 == 0 else (128 if N % 128 == 0 else N)
    nb = N // BT

    r2 = lambda b: b.reshape(1, -1)
    prep = pl.pallas_call(
        _prep_body,
        out_shape=(
            jax.ShapeDtypeStruct((N, 32), F32),
            jax.ShapeDtypeStruct((M, 32), F32),
        ),
    )
    Ep, Ev = prep(x_pfc, x_vtx, P1, r2(pb1), P2, r2(pb2), V1, r2(vb1), V2, r2(vb2))

    bt_p = batch_pfc.astype(jnp.int32).reshape(N, 1)
    bs_p = batch_pfc.astype(jnp.int32).reshape(1, N)
    bs_v = batch_vtx.astype(jnp.int32).reshape(1, M)
    wlo1, whi1 = _windows(batch_pfc, batch_pfc, BT, N // 128)
    wlo2, whi2 = _windows(batch_pfc, batch_vtx, BT, M // 128)

    smem = pl.BlockSpec(memory_space=pltpu.SMEM)
    full = lambda s: pl.BlockSpec(s, lambda i: (0, 0))
    blk = lambda s: pl.BlockSpec(s, lambda i: (i, 0))

    conv1 = pl.pallas_call(
        functools.partial(_conv1_body, BT=BT),
        grid=(nb,),
        in_specs=[smem, smem, blk((BT, 32)), blk((BT, 1)),
                  full((N, 32)), full((1, N)), full((64, 32)), full((1, 32))],
        out_specs=(blk((BT, 32)),),
        out_shape=(jax.ShapeDtypeStruct((N, 32), F32),),
        scratch_shapes=[pltpu.VMEM((BT, N), F32)],
    )
    (F1,) = conv1(wlo1, whi1, Ep, bt_p, Ep, bs_p, C1, r2(cb1))

    conv2 = pl.pallas_call(
        functools.partial(_conv2_body, BT=BT),
        grid=(nb,),
        in_specs=[smem, smem, blk((BT, 32)), blk((BT, 1)),
                  full((M, 32)), full((1, M)), full((64, 32)), full((1, 32)),
                  full((32, 64)), full((1, 64)), full((64, 32)), full((1, 32)),
                  full((32, 4)), full((1, 4)), full((4, 1)), full((1, 1))],
        out_specs=(blk((BT, 1)),),
        out_shape=(jax.ShapeDtypeStruct((N, 1), F32),),
        scratch_shapes=[pltpu.VMEM((BT, M), F32)],
    )
    (out,) = conv2(wlo2, whi2, F1, bt_p, Ev, bs_v, C1, r2(cb1),
                   O1, r2(ob1), O2, r2(ob2), O3, r2(ob3), O4, r2(ob4))
    return (out, batch_pfc)


# speedup vs baseline: 9.1532x; 1.0135x over previous
import functools
import jax
import jax.numpy as jnp
from jax import lax
from jax.experimental import pallas as pl
from jax.experimental.pallas import tpu as pltpu

F32 = jnp.float32
BIG1 = 1e10
BIG2 = 4e10
NEG = -1e30
HI = jax.lax.Precision.HIGHEST
K = 16
TW = 256


def _lr(x):
    return jnp.where(x > 0, x, 0.01 * x)


def _prep_body(xp_ref, xv_ref, P1_ref, pb1_ref, P2_ref, pb2_ref,
               V1_ref, vb1_ref, V2_ref, vb2_ref, Ep_ref, Ev_ref):
    xp = xp_ref[...]
    xv = xv_ref[...]
    Ep_ref[...] = _lr(jnp.dot(_lr(jnp.dot(xp, P1_ref[...]) + pb1_ref[...]),
                              P2_ref[...]) + pb2_ref[...])
    Ev_ref[...] = _lr(jnp.dot(_lr(jnp.dot(xv, V1_ref[...]) + vb1_ref[...]),
                              V2_ref[...]) + vb2_ref[...])


def _conv_core(A, bt, wlo, whi, b_ref, bs_ref, c1a, c1b, cb, strip_ref, BT):
    n_t = jnp.sum(A * A, axis=1, keepdims=True)
    ones_row = jnp.ones((1, 32), F32)
    ui = jnp.dot(A, c1a, preferred_element_type=F32)

    def p1(j, _):
        idx = pl.multiple_of(j * TW, TW)
        S_tile = b_ref[pl.ds(idx, TW), :]
        D0 = lax.dot_general(A, S_tile, (((1,), (1,)), ((), ())),
                             preferred_element_type=F32)
        ns = lax.dot_general(ones_row, S_tile * S_tile,
                             (((1,), (1,)), ((), ())),
                             precision=HI, preferred_element_type=F32)
        D = (n_t - 2.0 * D0) + ns
        bs = bs_ref[:, pl.ds(idx, TW)]
        D = jnp.where(bt != bs, BIG1, D)
        strip_ref[:, pl.ds(idx, TW)] = D
        return 0

    lax.fori_loop(wlo, whi, p1, 0)

    def sweep(s, carry):
        m_prev, acc = carry

        def inner(j, c):
            cm, xj = c
            idx = pl.multiple_of(j * TW, TW)
            t = strip_ref[:, pl.ds(idx, TW)]
            oh = t == m_prev
            t = jnp.where(oh, BIG2, t)
            strip_ref[:, pl.ds(idx, TW)] = t
            S_tile = b_ref[pl.ds(idx, TW), :]
            xj = xj + lax.dot_general(oh.astype(F32), S_tile,
                                      (((1,), (0,)), ((), ())),
                                      precision=HI, preferred_element_type=F32)
            return (jnp.minimum(cm, t), xj)

        cmv, XJ = lax.fori_loop(wlo, whi, inner,
                                (jnp.full((BT, TW), 1e30, F32),
                                 jnp.zeros((BT, 32), F32)))
        cm = jnp.min(cmv, axis=1, keepdims=True)
        msg = _lr((ui + jnp.dot(XJ - A, c1b, preferred_element_type=F32)) + cb)
        acc = jnp.where(s > 0, jnp.maximum(acc, msg), acc)
        return (cm, acc)

    _, acc = lax.fori_loop(0, K + 1, sweep,
                           (jnp.full((BT, 1), 3e30, F32),
                            jnp.full((BT, 32), NEG, F32)))
    return acc


def _conv1_body(wlo_ref, whi_ref, a_ref, bt_ref, b_ref, bs_ref,
                C1_ref, cb_ref, F_ref, strip_ref, *, BT):
    pid = pl.program_id(0)
    F_ref[...] = _conv_core(a_ref[...], bt_ref[...],
                            wlo_ref[pid], whi_ref[pid], b_ref, bs_ref,
                            C1_ref[0:32, :], C1_ref[32:64, :], cb_ref[...],
                            strip_ref, BT)


def _conv2_body(wlo_ref, whi_ref, a_ref, bt_ref, b_ref, bs_ref, C1_ref, cb_ref,
                O1_ref, ob1_ref, O2_ref, ob2_ref, O3_ref, ob3_ref,
                O4_ref, ob4_ref, out_ref, strip_ref, *, BT):
    pid = pl.program_id(0)
    feats = _conv_core(a_ref[...], bt_ref[...],
                       wlo_ref[pid], whi_ref[pid], b_ref, bs_ref,
                       C1_ref[0:32, :], C1_ref[32:64, :], cb_ref[...],
                       strip_ref, BT)
    h = _lr(jnp.dot(feats, O1_ref[...]) + ob1_ref[...])
    h = _lr(jnp.dot(h, O2_ref[...]) + ob2_ref[...])
    h = _lr(jnp.dot(h, O3_ref[...]) + ob3_ref[...])
    out_ref[...] = _lr(jnp.dot(h, O4_ref[...]) + ob4_ref[...])


def _windows(batch_t, batch_s, BT, n_blk_s):
    i0 = jnp.arange(batch_t.shape[0] // BT) * BT
    blo = batch_t[i0]
    bhi = batch_t[i0 + BT - 1]
    lo = jnp.searchsorted(batch_s, blo, side='left')
    hic = jnp.searchsorted(batch_s, bhi, side='right')
    wlo = (lo // TW).astype(jnp.int32)
    whi = jnp.minimum((hic + TW - 1) // TW, n_blk_s).astype(jnp.int32)
    whi = jnp.maximum(whi, wlo)
    return wlo, whi


def kernel(x_pfc, x_vtx, batch_pfc, batch_vtx,
           P1, pb1, P2, pb2, V1, vb1, V2, vb2, C1, cb1,
           O1, ob1, O2, ob2, O3, ob3, O4, ob4):
    N = x_pfc.shape[0]
    M = x_vtx.shape[0]
    BT = 512 if N % 512 == 0 else (128 if N % 128 == 0 else N)
    nb = N // BT

    r2 = lambda b: b.reshape(1, -1)
    prep = pl.pallas_call(
        _prep_body,
        out_shape=(
            jax.ShapeDtypeStruct((N, 32), F32),
            jax.ShapeDtypeStruct((M, 32), F32),
        ),
    )
    Ep, Ev = prep(x_pfc, x_vtx, P1, r2(pb1), P2, r2(pb2), V1, r2(vb1), V2, r2(vb2))

    bt_p = batch_pfc.astype(jnp.int32).reshape(N, 1)
    bs_p = batch_pfc.astype(jnp.int32).reshape(1, N)
    bs_v = batch_vtx.astype(jnp.int32).reshape(1, M)
    wlo1, whi1 = _windows(batch_pfc, batch_pfc, BT, N // TW)
    wlo2, whi2 = _windows(batch_pfc, batch_vtx, BT, M // TW)

    smem = pl.BlockSpec(memory_space=pltpu.SMEM)
    full = lambda s: pl.BlockSpec(s, lambda i: (0, 0))
    blk = lambda s: pl.BlockSpec(s, lambda i: (i, 0))

    conv1 = pl.pallas_call(
        functools.partial(_conv1_body, BT=BT),
        grid=(nb,),
        in_specs=[smem, smem, blk((BT, 32)), blk((BT, 1)),
                  full((N, 32)), full((1, N)), full((64, 32)), full((1, 32))],
        out_specs=(blk((BT, 32)),),
        out_shape=(jax.ShapeDtypeStruct((N, 32), F32),),
        scratch_shapes=[pltpu.VMEM((BT, N), F32)],
    )
    (F1,) = conv1(wlo1, whi1, Ep, bt_p, Ep, bs_p, C1, r2(cb1))

    conv2 = pl.pallas_call(
        functools.partial(_conv2_body, BT=BT),
        grid=(nb,),
        in_specs=[smem, smem, blk((BT, 32)), blk((BT, 1)),
                  full((M, 32)), full((1, M)), full((64, 32)), full((1, 32)),
                  full((32, 64)), full((1, 64)), full((64, 32)), full((1, 32)),
                  full((32, 4)), full((1, 4)), full((4, 1)), full((1, 1))],
        out_specs=(blk((BT, 1)),),
        out_shape=(jax.ShapeDtypeStruct((N, 1), F32),),
        scratch_shapes=[pltpu.VMEM((BT, M), F32)],
    )
    (out,) = conv2(wlo2, whi2, F1, bt_p, Ev, bs_v, C1, r2(cb1),
                   O1, r2(ob1), O2, r2(ob2), O3, r2(ob3), O4, r2(ob4))
    return (out, batch_pfc)


# speedup vs baseline: 10.7891x; 1.1787x over previous
import functools
import jax
import jax.numpy as jnp
from jax import lax
from jax.experimental import pallas as pl
from jax.experimental.pallas import tpu as pltpu

F32 = jnp.float32
BIG1 = 1e10
BIG2 = 4e10
NEG = -1e30
HI = jax.lax.Precision.HIGHEST
K = 16
TW = 256


def _lr(x):
    return jnp.where(x > 0, x, 0.01 * x)


def _prep_body(xp_ref, xv_ref, P1_ref, pb1_ref, P2_ref, pb2_ref,
               V1_ref, vb1_ref, V2_ref, vb2_ref, Ep_ref, Ev_ref):
    xp = xp_ref[...]
    xv = xv_ref[...]
    Ep_ref[...] = _lr(jnp.dot(_lr(jnp.dot(xp, P1_ref[...]) + pb1_ref[...]),
                              P2_ref[...]) + pb2_ref[...])
    Ev_ref[...] = _lr(jnp.dot(_lr(jnp.dot(xv, V1_ref[...]) + vb1_ref[...]),
                              V2_ref[...]) + vb2_ref[...])


def _conv_core(A, bt, wlo, whi, b_ref, bs_ref, c1a, c1b, cb, strip_ref, BT):
    n_t = jnp.sum(A * A, axis=1, keepdims=True)
    ones_row = jnp.ones((1, 32), F32)
    ui = jnp.dot(A, c1a, preferred_element_type=F32)

    def p1(j, _):
        idx = pl.multiple_of(j * TW, TW)
        S_tile = b_ref[pl.ds(idx, TW), :]
        D0 = lax.dot_general(A, S_tile, (((1,), (1,)), ((), ())),
                             preferred_element_type=F32)
        ns = lax.dot_general(ones_row, S_tile * S_tile,
                             (((1,), (1,)), ((), ())),
                             precision=HI, preferred_element_type=F32)
        D = (n_t - 2.0 * D0) + ns
        bs = bs_ref[:, pl.ds(idx, TW)]
        D = jnp.where(bt != bs, BIG1, D)
        strip_ref[:, pl.ds(idx, TW)] = D
        return 0

    lax.fori_loop(wlo, whi, p1, 0)

    m_list = []
    m_prev = jnp.full((BT, 1), -1e30, F32)
    for s in range(K):
        def sweep_inner(j, cmv, m_prev=m_prev):
            idx = pl.multiple_of(j * TW, TW)
            t = strip_ref[:, pl.ds(idx, TW)]
            return jnp.minimum(cmv, jnp.where(t > m_prev, t, BIG1))
        cmv = lax.fori_loop(wlo, whi, sweep_inner,
                            jnp.full((BT, TW), 1e30, F32))
        m_prev = jnp.min(cmv, axis=1, keepdims=True)
        m_list.append(m_prev)

    def pb(j, accs):
        idx = pl.multiple_of(j * TW, TW)
        t = strip_ref[:, pl.ds(idx, TW)]
        S_tile = b_ref[pl.ds(idx, TW), :]
        return tuple(
            accs[s] + lax.dot_general((t == m_list[s]).astype(F32), S_tile,
                                      (((1,), (0,)), ((), ())),
                                      precision=HI, preferred_element_type=F32)
            for s in range(K))

    XJs = lax.fori_loop(wlo, whi, pb,
                        tuple(jnp.zeros((BT, 32), F32) for _ in range(K)))
    acc = jnp.full((BT, 32), NEG, F32)
    for s in range(K):
        msg = _lr((ui + jnp.dot(XJs[s] - A, c1b, preferred_element_type=F32))
                  + cb)
        acc = jnp.maximum(acc, msg)
    return acc


def _conv1_body(wlo_ref, whi_ref, a_ref, bt_ref, b_ref, bs_ref,
                C1_ref, cb_ref, F_ref, strip_ref, *, BT):
    pid = pl.program_id(0)
    F_ref[...] = _conv_core(a_ref[...], bt_ref[...],
                            wlo_ref[pid], whi_ref[pid], b_ref, bs_ref,
                            C1_ref[0:32, :], C1_ref[32:64, :], cb_ref[...],
                            strip_ref, BT)


def _conv2_body(wlo_ref, whi_ref, a_ref, bt_ref, b_ref, bs_ref, C1_ref, cb_ref,
                O1_ref, ob1_ref, O2_ref, ob2_ref, O3_ref, ob3_ref,
                O4_ref, ob4_ref, out_ref, strip_ref, *, BT):
    pid = pl.program_id(0)
    feats = _conv_core(a_ref[...], bt_ref[...],
                       wlo_ref[pid], whi_ref[pid], b_ref, bs_ref,
                       C1_ref[0:32, :], C1_ref[32:64, :], cb_ref[...],
                       strip_ref, BT)
    h = _lr(jnp.dot(feats, O1_ref[...]) + ob1_ref[...])
    h = _lr(jnp.dot(h, O2_ref[...]) + ob2_ref[...])
    h = _lr(jnp.dot(h, O3_ref[...]) + ob3_ref[...])
    out_ref[...] = _lr(jnp.dot(h, O4_ref[...]) + ob4_ref[...])


def _windows(batch_t, batch_s, BT, n_blk_s):
    i0 = jnp.arange(batch_t.shape[0] // BT) * BT
    blo = batch_t[i0]
    bhi = batch_t[i0 + BT - 1]
    lo = jnp.searchsorted(batch_s, blo, side='left')
    hic = jnp.searchsorted(batch_s, bhi, side='right')
    wlo = (lo // TW).astype(jnp.int32)
    whi = jnp.minimum((hic + TW - 1) // TW, n_blk_s).astype(jnp.int32)
    whi = jnp.maximum(whi, wlo)
    return wlo, whi


def kernel(x_pfc, x_vtx, batch_pfc, batch_vtx,
           P1, pb1, P2, pb2, V1, vb1, V2, vb2, C1, cb1,
           O1, ob1, O2, ob2, O3, ob3, O4, ob4):
    N = x_pfc.shape[0]
    M = x_vtx.shape[0]
    BT = 512 if N % 512 == 0 else (128 if N % 128 == 0 else N)
    nb = N // BT

    r2 = lambda b: b.reshape(1, -1)
    prep = pl.pallas_call(
        _prep_body,
        out_shape=(
            jax.ShapeDtypeStruct((N, 32), F32),
            jax.ShapeDtypeStruct((M, 32), F32),
        ),
    )
    Ep, Ev = prep(x_pfc, x_vtx, P1, r2(pb1), P2, r2(pb2), V1, r2(vb1), V2, r2(vb2))

    bt_p = batch_pfc.astype(jnp.int32).reshape(N, 1)
    bs_p = batch_pfc.astype(jnp.int32).reshape(1, N)
    bs_v = batch_vtx.astype(jnp.int32).reshape(1, M)
    wlo1, whi1 = _windows(batch_pfc, batch_pfc, BT, N // TW)
    wlo2, whi2 = _windows(batch_pfc, batch_vtx, BT, M // TW)

    smem = pl.BlockSpec(memory_space=pltpu.SMEM)
    full = lambda s: pl.BlockSpec(s, lambda i: (0, 0))
    blk = lambda s: pl.BlockSpec(s, lambda i: (i, 0))

    conv1 = pl.pallas_call(
        functools.partial(_conv1_body, BT=BT),
        grid=(nb,),
        in_specs=[smem, smem, blk((BT, 32)), blk((BT, 1)),
                  full((N, 32)), full((1, N)), full((64, 32)), full((1, 32))],
        out_specs=(blk((BT, 32)),),
        out_shape=(jax.ShapeDtypeStruct((N, 32), F32),),
        scratch_shapes=[pltpu.VMEM((BT, N), F32)],
    )
    (F1,) = conv1(wlo1, whi1, Ep, bt_p, Ep, bs_p, C1, r2(cb1))

    conv2 = pl.pallas_call(
        functools.partial(_conv2_body, BT=BT),
        grid=(nb,),
        in_specs=[smem, smem, blk((BT, 32)), blk((BT, 1)),
                  full((M, 32)), full((1, M)), full((64, 32)), full((1, 32)),
                  full((32, 64)), full((1, 64)), full((64, 32)), full((1, 32)),
                  full((32, 4)), full((1, 4)), full((4, 1)), full((1, 1))],
        out_specs=(blk((BT, 1)),),
        out_shape=(jax.ShapeDtypeStruct((N, 1), F32),),
        scratch_shapes=[pltpu.VMEM((BT, M), F32)],
    )
    (out,) = conv2(wlo2, whi2, F1, bt_p, Ev, bs_v, C1, r2(cb1),
                   O1, r2(ob1), O2, r2(ob2), O3, r2(ob3), O4, r2(ob4))
    return (out, batch_pfc)


# speedup vs baseline: 16.5000x; 1.5293x over previous
import functools
import jax
import jax.numpy as jnp
from jax import lax
from jax.experimental import pallas as pl
from jax.experimental.pallas import tpu as pltpu

F32 = jnp.float32
BIG1 = 1e10
BIG2 = 4e10
NEG = -1e30
HI = jax.lax.Precision.HIGHEST
K = 16
TW = 256


def _lr(x):
    return jnp.where(x > 0, x, 0.01 * x)


def _split3(x):
    s1 = x.astype(jnp.bfloat16).astype(F32)
    r = x - s1
    s2 = r.astype(jnp.bfloat16).astype(F32)
    s3 = r - s2
    return jnp.concatenate([s1, s2, s3], axis=1)


def _prep_body(xp_ref, xv_ref, P1_ref, pb1_ref, P2_ref, pb2_ref,
               V1_ref, vb1_ref, V2_ref, vb2_ref,
               Ep_ref, Ev_ref, Ep3_ref, Ev3_ref):
    xp = xp_ref[...]
    xv = xv_ref[...]
    ep = _lr(jnp.dot(_lr(jnp.dot(xp, P1_ref[...]) + pb1_ref[...]),
                     P2_ref[...]) + pb2_ref[...])
    ev = _lr(jnp.dot(_lr(jnp.dot(xv, V1_ref[...]) + vb1_ref[...]),
                     V2_ref[...]) + vb2_ref[...])
    Ep_ref[...] = ep
    Ev_ref[...] = ev
    Ep3_ref[...] = _split3(ep)
    Ev3_ref[...] = _split3(ev)


def _conv_core(A, bt, wlo, whi, b_ref, b3_ref, bs_ref, c1a, c1b, cb, strip_ref, BT):
    n_t = jnp.sum(A * A, axis=1, keepdims=True)
    ones_row = jnp.ones((1, 32), F32)
    ui = jnp.dot(A, c1a, preferred_element_type=F32)

    def p1(j, _):
        idx = pl.multiple_of(j * TW, TW)
        S_tile = b_ref[pl.ds(idx, TW), :]
        D0 = lax.dot_general(A, S_tile, (((1,), (1,)), ((), ())),
                             preferred_element_type=F32)
        ns = lax.dot_general(ones_row, S_tile * S_tile,
                             (((1,), (1,)), ((), ())),
                             precision=HI, preferred_element_type=F32)
        D = (n_t - 2.0 * D0) + ns
        bs = bs_ref[:, pl.ds(idx, TW)]
        D = jnp.where(bt != bs, BIG1, D)
        strip_ref[:, pl.ds(idx, TW)] = D
        return 0

    lax.fori_loop(wlo, whi, p1, 0)

    m_list = []
    m_prev = jnp.full((BT, 1), -1e30, F32)
    for s in range(K):
        def sweep_inner(j, cmv, m_prev=m_prev):
            idx = pl.multiple_of(j * TW, TW)
            t = strip_ref[:, pl.ds(idx, TW)]
            return jnp.minimum(cmv, jnp.where(t > m_prev, t, BIG1))
        cmv = lax.fori_loop(wlo, whi, sweep_inner,
                            jnp.full((BT, TW), 1e30, F32))
        m_prev = jnp.min(cmv, axis=1, keepdims=True)
        m_list.append(m_prev)

    def pb(j, accs):
        idx = pl.multiple_of(j * TW, TW)
        t = strip_ref[:, pl.ds(idx, TW)]
        S3_tile = b3_ref[pl.ds(idx, TW), :]
        new = []
        for s in range(K):
            g = lax.dot_general((t == m_list[s]).astype(F32), S3_tile,
                                (((1,), (0,)), ((), ())),
                                preferred_element_type=F32)
            new.append(accs[s] + ((g[:, 0:32] + g[:, 32:64]) + g[:, 64:96]))
        return tuple(new)

    XJs = lax.fori_loop(wlo, whi, pb,
                        tuple(jnp.zeros((BT, 32), F32) for _ in range(K)))
    acc = jnp.full((BT, 32), NEG, F32)
    for s in range(K):
        msg = _lr((ui + jnp.dot(XJs[s] - A, c1b, preferred_element_type=F32))
                  + cb)
        acc = jnp.maximum(acc, msg)
    return acc


def _conv1_body(wlo_ref, whi_ref, a_ref, bt_ref, b_ref, b3_ref, bs_ref,
                C1_ref, cb_ref, F_ref, F3_ref, strip_ref, *, BT):
    pid = pl.program_id(0)
    feats = _conv_core(a_ref[...], bt_ref[...],
                       wlo_ref[pid], whi_ref[pid], b_ref, b3_ref, bs_ref,
                       C1_ref[0:32, :], C1_ref[32:64, :], cb_ref[...],
                       strip_ref, BT)
    F_ref[...] = feats
    F3_ref[...] = _split3(feats)


def _conv2_body(wlo_ref, whi_ref, a_ref, bt_ref, b_ref, b3_ref, bs_ref, C1_ref, cb_ref,
                O1_ref, ob1_ref, O2_ref, ob2_ref, O3_ref, ob3_ref,
                O4_ref, ob4_ref, out_ref, strip_ref, *, BT):
    pid = pl.program_id(0)
    feats = _conv_core(a_ref[...], bt_ref[...],
                       wlo_ref[pid], whi_ref[pid], b_ref, b3_ref, bs_ref,
                       C1_ref[0:32, :], C1_ref[32:64, :], cb_ref[...],
                       strip_ref, BT)
    h = _lr(jnp.dot(feats, O1_ref[...]) + ob1_ref[...])
    h = _lr(jnp.dot(h, O2_ref[...]) + ob2_ref[...])
    h = _lr(jnp.dot(h, O3_ref[...]) + ob3_ref[...])
    out_ref[...] = _lr(jnp.dot(h, O4_ref[...]) + ob4_ref[...])


def _windows(batch_t, batch_s, BT, n_blk_s):
    i0 = jnp.arange(batch_t.shape[0] // BT) * BT
    blo = batch_t[i0]
    bhi = batch_t[i0 + BT - 1]
    lo = jnp.searchsorted(batch_s, blo, side='left')
    hic = jnp.searchsorted(batch_s, bhi, side='right')
    wlo = (lo // TW).astype(jnp.int32)
    whi = jnp.minimum((hic + TW - 1) // TW, n_blk_s).astype(jnp.int32)
    whi = jnp.maximum(whi, wlo)
    return wlo, whi


def kernel(x_pfc, x_vtx, batch_pfc, batch_vtx,
           P1, pb1, P2, pb2, V1, vb1, V2, vb2, C1, cb1,
           O1, ob1, O2, ob2, O3, ob3, O4, ob4):
    N = x_pfc.shape[0]
    M = x_vtx.shape[0]
    BT = 512 if N % 512 == 0 else (128 if N % 128 == 0 else N)
    nb = N // BT

    r2 = lambda b: b.reshape(1, -1)
    prep = pl.pallas_call(
        _prep_body,
        out_shape=(
            jax.ShapeDtypeStruct((N, 32), F32),
            jax.ShapeDtypeStruct((M, 32), F32),
            jax.ShapeDtypeStruct((N, 96), F32),
            jax.ShapeDtypeStruct((M, 96), F32),
        ),
    )
    Ep, Ev, Ep3, Ev3 = prep(x_pfc, x_vtx, P1, r2(pb1), P2, r2(pb2), V1, r2(vb1), V2, r2(vb2))

    bt_p = batch_pfc.astype(jnp.int32).reshape(N, 1)
    bs_p = batch_pfc.astype(jnp.int32).reshape(1, N)
    bs_v = batch_vtx.astype(jnp.int32).reshape(1, M)
    wlo1, whi1 = _windows(batch_pfc, batch_pfc, BT, N // TW)
    wlo2, whi2 = _windows(batch_pfc, batch_vtx, BT, M // TW)

    smem = pl.BlockSpec(memory_space=pltpu.SMEM)
    full = lambda s: pl.BlockSpec(s, lambda i: (0, 0))
    blk = lambda s: pl.BlockSpec(s, lambda i: (i, 0))

    conv1 = pl.pallas_call(
        functools.partial(_conv1_body, BT=BT),
        grid=(nb,),
        in_specs=[smem, smem, blk((BT, 32)), blk((BT, 1)),
                  full((N, 32)), full((N, 96)), full((1, N)),
                  full((64, 32)), full((1, 32))],
        out_specs=(blk((BT, 32)), blk((BT, 96))),
        out_shape=(jax.ShapeDtypeStruct((N, 32), F32),
                   jax.ShapeDtypeStruct((N, 96), F32)),
        scratch_shapes=[pltpu.VMEM((BT, N), F32)],
    )
    F1, F13 = conv1(wlo1, whi1, Ep, bt_p, Ep, Ep3, bs_p, C1, r2(cb1))

    conv2 = pl.pallas_call(
        functools.partial(_conv2_body, BT=BT),
        grid=(nb,),
        in_specs=[smem, smem, blk((BT, 32)), blk((BT, 1)),
                  full((M, 32)), full((M, 96)), full((1, M)),
                  full((64, 32)), full((1, 32)),
                  full((32, 64)), full((1, 64)), full((64, 32)), full((1, 32)),
                  full((32, 4)), full((1, 4)), full((4, 1)), full((1, 1))],
        out_specs=(blk((BT, 1)),),
        out_shape=(jax.ShapeDtypeStruct((N, 1), F32),),
        scratch_shapes=[pltpu.VMEM((BT, M), F32)],
    )
    (out,) = conv2(wlo2, whi2, F1, bt_p, Ev, Ev3, bs_v, C1, r2(cb1),
                   O1, r2(ob1), O2, r2(ob2), O3, r2(ob3), O4, r2(ob4))
    return (out, batch_pfc)


# speedup vs baseline: 16.8280x; 1.0199x over previous
import functools
import jax
import jax.numpy as jnp
from jax import lax
from jax.experimental import pallas as pl
from jax.experimental.pallas import tpu as pltpu

F32 = jnp.float32
BIG1 = 1e10
BIG2 = 4e10
NEG = -1e30
HI = jax.lax.Precision.HIGHEST
K = 16
TW = 256


def _lr(x):
    return jnp.where(x > 0, x, 0.01 * x)


def _split3(x):
    s1 = x.astype(jnp.bfloat16).astype(F32)
    r = x - s1
    s2 = r.astype(jnp.bfloat16).astype(F32)
    s3 = r - s2
    return jnp.concatenate([s1, s2, s3], axis=1)


def _prep_body(xp_ref, xv_ref, P1_ref, pb1_ref, P2_ref, pb2_ref,
               V1_ref, vb1_ref, V2_ref, vb2_ref,
               Ep_ref, Ev_ref, Ep3_ref, Ev3_ref):
    xp = xp_ref[...]
    xv = xv_ref[...]
    ep = _lr(jnp.dot(_lr(jnp.dot(xp, P1_ref[...]) + pb1_ref[...]),
                     P2_ref[...]) + pb2_ref[...])
    ev = _lr(jnp.dot(_lr(jnp.dot(xv, V1_ref[...]) + vb1_ref[...]),
                     V2_ref[...]) + vb2_ref[...])
    Ep_ref[...] = ep
    Ev_ref[...] = ev
    Ep3_ref[...] = _split3(ep)
    Ev3_ref[...] = _split3(ev)


def _conv_core(A, bt, wlo, whi, b_ref, b3_ref, bs_ref, c1a, c1b, cb, strip_ref, BT):
    n_t = jnp.sum(A * A, axis=1, keepdims=True)
    ones_row = jnp.ones((1, 32), F32)
    ui = jnp.dot(A, c1a, preferred_element_type=F32)

    def p1(j, _):
        idx = pl.multiple_of(j * TW, TW)
        S_tile = b_ref[pl.ds(idx, TW), :]
        D0 = lax.dot_general(A, S_tile, (((1,), (1,)), ((), ())),
                             preferred_element_type=F32)
        ns = lax.dot_general(ones_row, S_tile * S_tile,
                             (((1,), (1,)), ((), ())),
                             precision=HI, preferred_element_type=F32)
        D = (n_t - 2.0 * D0) + ns
        bs = bs_ref[:, pl.ds(idx, TW)]
        D = jnp.where(bt != bs, BIG1, D)
        strip_ref[:, pl.ds(idx, TW)] = D
        return 0

    lax.fori_loop(wlo, whi, p1, 0)

    m_list = []
    m_prev = jnp.full((BT, 1), -1e30, F32)
    for s in range(K):
        def sweep_inner(j, cmv, m_prev=m_prev):
            idx = pl.multiple_of(j * TW, TW)
            t = strip_ref[:, pl.ds(idx, TW)]
            return jnp.minimum(cmv, jnp.where(t > m_prev, t, BIG1))
        cmv = lax.fori_loop(wlo, whi, sweep_inner,
                            jnp.full((BT, TW), 1e30, F32))
        m_prev = jnp.min(cmv, axis=1, keepdims=True)
        m_list.append(m_prev)

    def pb(j, accs):
        idx = pl.multiple_of(j * TW, TW)
        t = strip_ref[:, pl.ds(idx, TW)]
        S3_tile = b3_ref[pl.ds(idx, TW), :]
        new = []
        for s in range(K):
            g = lax.dot_general((t == m_list[s]).astype(F32), S3_tile,
                                (((1,), (0,)), ((), ())),
                                preferred_element_type=F32)
            new.append(accs[s] + ((g[:, 0:32] + g[:, 32:64]) + g[:, 64:96]))
        return tuple(new)

    XJs = lax.fori_loop(wlo, whi, pb,
                        tuple(jnp.zeros((BT, 32), F32) for _ in range(K)))
    acc = jnp.full((BT, 32), NEG, F32)
    for s in range(K):
        msg = _lr((ui + jnp.dot(XJs[s] - A, c1b, preferred_element_type=F32))
                  + cb)
        acc = jnp.maximum(acc, msg)
    return acc


def _conv1_body(wlo_ref, whi_ref, a_ref, bt_ref, b_ref, b3_ref, bs_ref,
                C1_ref, cb_ref, F_ref, F3_ref, strip_ref, *, BT):
    pid = pl.program_id(0)
    feats = _conv_core(a_ref[...], bt_ref[...],
                       wlo_ref[pid], whi_ref[pid], b_ref, b3_ref, bs_ref,
                       C1_ref[0:32, :], C1_ref[32:64, :], cb_ref[...],
                       strip_ref, BT)
    F_ref[...] = feats
    F3_ref[...] = _split3(feats)


def _conv2_body(wlo_ref, whi_ref, a_ref, bt_ref, b_ref, b3_ref, bs_ref, C1_ref, cb_ref,
                O1_ref, ob1_ref, O2_ref, ob2_ref, O3_ref, ob3_ref,
                O4_ref, ob4_ref, out_ref, strip_ref, *, BT):
    pid = pl.program_id(0)
    feats = _conv_core(a_ref[...], bt_ref[...],
                       wlo_ref[pid], whi_ref[pid], b_ref, b3_ref, bs_ref,
                       C1_ref[0:32, :], C1_ref[32:64, :], cb_ref[...],
                       strip_ref, BT)
    h = _lr(jnp.dot(feats, O1_ref[...]) + ob1_ref[...])
    h = _lr(jnp.dot(h, O2_ref[...]) + ob2_ref[...])
    h = _lr(jnp.dot(h, O3_ref[...]) + ob3_ref[...])
    out_ref[...] = _lr(jnp.dot(h, O4_ref[...]) + ob4_ref[...])


def _windows(batch_t, batch_s, BT, n_blk_s):
    i0 = jnp.arange(batch_t.shape[0] // BT) * BT
    blo = batch_t[i0]
    bhi = batch_t[i0 + BT - 1]
    lo = jnp.searchsorted(batch_s, blo, side='left')
    hic = jnp.searchsorted(batch_s, bhi, side='right')
    wlo = (lo // TW).astype(jnp.int32)
    whi = jnp.minimum((hic + TW - 1) // TW, n_blk_s).astype(jnp.int32)
    whi = jnp.maximum(whi, wlo)
    return wlo, whi


def kernel(x_pfc, x_vtx, batch_pfc, batch_vtx,
           P1, pb1, P2, pb2, V1, vb1, V2, vb2, C1, cb1,
           O1, ob1, O2, ob2, O3, ob3, O4, ob4):
    N = x_pfc.shape[0]
    M = x_vtx.shape[0]
    BT = 128 if N % 128 == 0 else N
    nb = N // BT

    r2 = lambda b: b.reshape(1, -1)
    prep = pl.pallas_call(
        _prep_body,
        out_shape=(
            jax.ShapeDtypeStruct((N, 32), F32),
            jax.ShapeDtypeStruct((M, 32), F32),
            jax.ShapeDtypeStruct((N, 96), F32),
            jax.ShapeDtypeStruct((M, 96), F32),
        ),
    )
    Ep, Ev, Ep3, Ev3 = prep(x_pfc, x_vtx, P1, r2(pb1), P2, r2(pb2), V1, r2(vb1), V2, r2(vb2))

    bt_p = batch_pfc.astype(jnp.int32).reshape(N, 1)
    bs_p = batch_pfc.astype(jnp.int32).reshape(1, N)
    bs_v = batch_vtx.astype(jnp.int32).reshape(1, M)
    wlo1, whi1 = _windows(batch_pfc, batch_pfc, BT, N // TW)
    wlo2, whi2 = _windows(batch_pfc, batch_vtx, BT, M // TW)

    smem = pl.BlockSpec(memory_space=pltpu.SMEM)
    full = lambda s: pl.BlockSpec(s, lambda i: (0, 0))
    blk = lambda s: pl.BlockSpec(s, lambda i: (i, 0))

    conv1 = pl.pallas_call(
        functools.partial(_conv1_body, BT=BT),
        grid=(nb,),
        in_specs=[smem, smem, blk((BT, 32)), blk((BT, 1)),
                  full((N, 32)), full((N, 96)), full((1, N)),
                  full((64, 32)), full((1, 32))],
        out_specs=(blk((BT, 32)), blk((BT, 96))),
        out_shape=(jax.ShapeDtypeStruct((N, 32), F32),
                   jax.ShapeDtypeStruct((N, 96), F32)),
        scratch_shapes=[pltpu.VMEM((BT, N), F32)],
    )
    F1, F13 = conv1(wlo1, whi1, Ep, bt_p, Ep, Ep3, bs_p, C1, r2(cb1))

    conv2 = pl.pallas_call(
        functools.partial(_conv2_body, BT=BT),
        grid=(nb,),
        in_specs=[smem, smem, blk((BT, 32)), blk((BT, 1)),
                  full((M, 32)), full((M, 96)), full((1, M)),
                  full((64, 32)), full((1, 32)),
                  full((32, 64)), full((1, 64)), full((64, 32)), full((1, 32)),
                  full((32, 4)), full((1, 4)), full((4, 1)), full((1, 1))],
        out_specs=(blk((BT, 1)),),
        out_shape=(jax.ShapeDtypeStruct((N, 1), F32),),
        scratch_shapes=[pltpu.VMEM((BT, M), F32)],
    )
    (out,) = conv2(wlo2, whi2, F1, bt_p, Ev, Ev3, bs_v, C1, r2(cb1),
                   O1, r2(ob1), O2, r2(ob2), O3, r2(ob3), O4, r2(ob4))
    return (out, batch_pfc)


# speedup vs baseline: 16.8818x; 1.0032x over previous
import functools
import jax
import jax.numpy as jnp
from jax import lax
from jax.experimental import pallas as pl
from jax.experimental.pallas import tpu as pltpu

F32 = jnp.float32
BIG1 = 1e10
BIG2 = 4e10
NEG = -1e30
HI = jax.lax.Precision.HIGHEST
K = 16
TW = 256


def _lr(x):
    return jnp.where(x > 0, x, 0.01 * x)


def _split3(x):
    s1 = x.astype(jnp.bfloat16).astype(F32)
    r = x - s1
    s2 = r.astype(jnp.bfloat16).astype(F32)
    s3 = r - s2
    return jnp.concatenate([s1, s2, s3], axis=1)


def _prep_body(xp_ref, xv_ref, P1_ref, pb1_ref, P2_ref, pb2_ref,
               V1_ref, vb1_ref, V2_ref, vb2_ref,
               Ep_ref, Ev_ref, Ep3_ref, Ev3_ref):
    xp = xp_ref[...]
    xv = xv_ref[...]
    ep = _lr(jnp.dot(_lr(jnp.dot(xp, P1_ref[...]) + pb1_ref[...]),
                     P2_ref[...]) + pb2_ref[...])
    ev = _lr(jnp.dot(_lr(jnp.dot(xv, V1_ref[...]) + vb1_ref[...]),
                     V2_ref[...]) + vb2_ref[...])
    Ep_ref[...] = ep
    Ev_ref[...] = ev
    Ep3_ref[...] = _split3(ep)
    Ev3_ref[...] = _split3(ev)


def _conv_core(A, bt, wlo, whi, b_ref, b3_ref, bs_ref, c1a, c1b, cb, strip_ref, BT):
    n_t = jnp.sum(A * A, axis=1, keepdims=True)
    ones_row = jnp.ones((1, 32), F32)
    ui = jnp.dot(A, c1a, preferred_element_type=F32)

    def p1(j, _):
        idx = pl.multiple_of(j * TW, TW)
        S_tile = b_ref[pl.ds(idx, TW), :]
        D0 = lax.dot_general(A, S_tile, (((1,), (1,)), ((), ())),
                             preferred_element_type=F32)
        ns = lax.dot_general(ones_row, S_tile * S_tile,
                             (((1,), (1,)), ((), ())),
                             precision=HI, preferred_element_type=F32)
        D = (n_t - 2.0 * D0) + ns
        bs = bs_ref[:, pl.ds(idx, TW)]
        D = jnp.where(bt != bs, BIG1, D)
        strip_ref[:, pl.ds(idx, TW)] = D
        return jnp.minimum(_, D)

    cmv0 = lax.fori_loop(wlo, whi, p1, jnp.full((BT, TW), 1e30, F32))

    m_prev = jnp.min(cmv0, axis=1, keepdims=True)
    m_list = [m_prev]
    for s in range(K - 1):
        def sweep_inner(j, cmv, m_prev=m_prev):
            idx = pl.multiple_of(j * TW, TW)
            t = strip_ref[:, pl.ds(idx, TW)]
            return jnp.minimum(cmv, jnp.where(t > m_prev, t, BIG1))
        cmv = lax.fori_loop(wlo, whi, sweep_inner,
                            jnp.full((BT, TW), 1e30, F32))
        m_prev = jnp.min(cmv, axis=1, keepdims=True)
        m_list.append(m_prev)

    def pb(j, accs):
        idx = pl.multiple_of(j * TW, TW)
        t = strip_ref[:, pl.ds(idx, TW)]
        S3_tile = b3_ref[pl.ds(idx, TW), :]
        new = []
        for s in range(K):
            g = lax.dot_general((t == m_list[s]).astype(F32), S3_tile,
                                (((1,), (0,)), ((), ())),
                                preferred_element_type=F32)
            new.append(accs[s] + ((g[:, 0:32] + g[:, 32:64]) + g[:, 64:96]))
        return tuple(new)

    XJs = lax.fori_loop(wlo, whi, pb,
                        tuple(jnp.zeros((BT, 32), F32) for _ in range(K)))
    acc = jnp.full((BT, 32), NEG, F32)
    for s in range(K):
        msg = _lr((ui + jnp.dot(XJs[s] - A, c1b, preferred_element_type=F32))
                  + cb)
        acc = jnp.maximum(acc, msg)
    return acc


def _conv1_body(wlo_ref, whi_ref, a_ref, bt_ref, b_ref, b3_ref, bs_ref,
                C1_ref, cb_ref, F_ref, F3_ref, strip_ref, *, BT):
    pid = pl.program_id(0)
    feats = _conv_core(a_ref[...], bt_ref[...],
                       wlo_ref[pid], whi_ref[pid], b_ref, b3_ref, bs_ref,
                       C1_ref[0:32, :], C1_ref[32:64, :], cb_ref[...],
                       strip_ref, BT)
    F_ref[...] = feats
    F3_ref[...] = _split3(feats)


def _conv2_body(wlo_ref, whi_ref, a_ref, bt_ref, b_ref, b3_ref, bs_ref, C1_ref, cb_ref,
                O1_ref, ob1_ref, O2_ref, ob2_ref, O3_ref, ob3_ref,
                O4_ref, ob4_ref, out_ref, strip_ref, *, BT):
    pid = pl.program_id(0)
    feats = _conv_core(a_ref[...], bt_ref[...],
                       wlo_ref[pid], whi_ref[pid], b_ref, b3_ref, bs_ref,
                       C1_ref[0:32, :], C1_ref[32:64, :], cb_ref[...],
                       strip_ref, BT)
    h = _lr(jnp.dot(feats, O1_ref[...]) + ob1_ref[...])
    h = _lr(jnp.dot(h, O2_ref[...]) + ob2_ref[...])
    h = _lr(jnp.dot(h, O3_ref[...]) + ob3_ref[...])
    out_ref[...] = _lr(jnp.dot(h, O4_ref[...]) + ob4_ref[...])


def _windows(batch_t, batch_s, BT, n_blk_s):
    i0 = jnp.arange(batch_t.shape[0] // BT) * BT
    blo = batch_t[i0]
    bhi = batch_t[i0 + BT - 1]
    lo = jnp.searchsorted(batch_s, blo, side='left')
    hic = jnp.searchsorted(batch_s, bhi, side='right')
    wlo = (lo // TW).astype(jnp.int32)
    whi = jnp.minimum((hic + TW - 1) // TW, n_blk_s).astype(jnp.int32)
    whi = jnp.maximum(whi, wlo)
    return wlo, whi


def kernel(x_pfc, x_vtx, batch_pfc, batch_vtx,
           P1, pb1, P2, pb2, V1, vb1, V2, vb2, C1, cb1,
           O1, ob1, O2, ob2, O3, ob3, O4, ob4):
    N = x_pfc.shape[0]
    M = x_vtx.shape[0]
    BT = 128 if N % 128 == 0 else N
    nb = N // BT

    r2 = lambda b: b.reshape(1, -1)
    prep = pl.pallas_call(
        _prep_body,
        out_shape=(
            jax.ShapeDtypeStruct((N, 32), F32),
            jax.ShapeDtypeStruct((M, 32), F32),
            jax.ShapeDtypeStruct((N, 96), F32),
            jax.ShapeDtypeStruct((M, 96), F32),
        ),
    )
    Ep, Ev, Ep3, Ev3 = prep(x_pfc, x_vtx, P1, r2(pb1), P2, r2(pb2), V1, r2(vb1), V2, r2(vb2))

    bt_p = batch_pfc.astype(jnp.int32).reshape(N, 1)
    bs_p = batch_pfc.astype(jnp.int32).reshape(1, N)
    bs_v = batch_vtx.astype(jnp.int32).reshape(1, M)
    wlo1, whi1 = _windows(batch_pfc, batch_pfc, BT, N // TW)
    wlo2, whi2 = _windows(batch_pfc, batch_vtx, BT, M // TW)

    smem = pl.BlockSpec(memory_space=pltpu.SMEM)
    full = lambda s: pl.BlockSpec(s, lambda i: (0, 0))
    blk = lambda s: pl.BlockSpec(s, lambda i: (i, 0))

    conv1 = pl.pallas_call(
        functools.partial(_conv1_body, BT=BT),
        grid=(nb,),
        in_specs=[smem, smem, blk((BT, 32)), blk((BT, 1)),
                  full((N, 32)), full((N, 96)), full((1, N)),
                  full((64, 32)), full((1, 32))],
        out_specs=(blk((BT, 32)), blk((BT, 96))),
        out_shape=(jax.ShapeDtypeStruct((N, 32), F32),
                   jax.ShapeDtypeStruct((N, 96), F32)),
        scratch_shapes=[pltpu.VMEM((BT, N), F32)],
    )
    F1, F13 = conv1(wlo1, whi1, Ep, bt_p, Ep, Ep3, bs_p, C1, r2(cb1))

    conv2 = pl.pallas_call(
        functools.partial(_conv2_body, BT=BT),
        grid=(nb,),
        in_specs=[smem, smem, blk((BT, 32)), blk((BT, 1)),
                  full((M, 32)), full((M, 96)), full((1, M)),
                  full((64, 32)), full((1, 32)),
                  full((32, 64)), full((1, 64)), full((64, 32)), full((1, 32)),
                  full((32, 4)), full((1, 4)), full((4, 1)), full((1, 1))],
        out_specs=(blk((BT, 1)),),
        out_shape=(jax.ShapeDtypeStruct((N, 1), F32),),
        scratch_shapes=[pltpu.VMEM((BT, M), F32)],
    )
    (out,) = conv2(wlo2, whi2, F1, bt_p, Ev, Ev3, bs_v, C1, r2(cb1),
                   O1, r2(ob1), O2, r2(ob2), O3, r2(ob3), O4, r2(ob4))
    return (out, batch_pfc)
